# Initial kernel scaffold; baseline (speedup 1.0000x reference)
import jax
import jax.numpy as jnp
from jax.experimental import pallas as pl


def kernel(x, edge_index, W1, b1, W2, b2, Wd1, bd1, Wd2, bd2, Wfc, bfc):
    raise NotImplementedError("write your pallas kernel here")


# speedup vs baseline: 2.9642x; 2.9642x over previous
import functools

import jax
import jax.numpy as jnp
from jax import lax
from jax.experimental import pallas as pl
from jax.experimental.pallas import tpu as pltpu
from jax.experimental.pallas import tpu_sc as plsc

F32 = jnp.float32
_NS = 16
_NC = 2
_CH = 128


@functools.lru_cache(maxsize=None)
def _deg_kernel(n_pad, e_pad, w):
    rows_pt = n_pad // _NS
    edges_pt = e_pad // _NS
    nchunk = edges_pt // _CH
    mesh = plsc.VectorSubcoreMesh(core_axis_name="c", subcore_axis_name="s")

    @functools.partial(
        pl.kernel,
        mesh=mesh,
        out_type=jax.ShapeDtypeStruct((_NC * n_pad, w), F32),
        scratch_types=[pltpu.VMEM((_CH,), jnp.int32),
                       pltpu.VMEM((_CH, w), F32),
                       pltpu.VMEM_SHARED((n_pad, w), F32)],
    )
    def deg(idx2, zeros16, ones16, hist2, idx_v, ones_v, hist):
        c = lax.axis_index("c")
        s = lax.axis_index("s")
        pltpu.sync_copy(zeros16, hist.at[pl.ds(s * rows_pt, rows_pt)])
        pltpu.sync_copy(ones16, ones_v)
        plsc.subcore_barrier()
        base = c * e_pad + s * edges_pt

        def chunk(j, carry):
            pltpu.sync_copy(idx2.at[pl.ds(base + j * _CH, _CH)], idx_v)
            pltpu.sync_copy(ones_v, hist.at[idx_v], add=True)
            return carry

        lax.fori_loop(0, nchunk, chunk, 0)
        plsc.subcore_barrier()
        sl = pl.ds(s * rows_pt, rows_pt)
        pltpu.sync_copy(hist.at[sl],
                        hist2.at[pl.ds(c * n_pad + s * rows_pt, rows_pt)])

    return deg


@functools.lru_cache(maxsize=None)
def _agg_kernel(n_pad, e_pad, h):
    rows_pt = n_pad // _NS
    edges_pt = e_pad // _NS
    nchunk = edges_pt // _CH
    mesh = plsc.VectorSubcoreMesh(core_axis_name="c", subcore_axis_name="s")

    @functools.partial(
        pl.kernel,
        mesh=mesh,
        out_type=jax.ShapeDtypeStruct((_NC * n_pad, h), F32),
        scratch_types=[pltpu.VMEM((_CH,), jnp.int32),
                       pltpu.VMEM((_CH,), jnp.int32),
                       pltpu.VMEM((_CH, h), F32),
                       pltpu.VMEM_SHARED((n_pad, h), F32),
                       pltpu.SemaphoreType.DMA],
    )
    def agg(table2, src2, dst_h, zrow, out2, idx_s, idx_d, rows, acc, sem):
        c = lax.axis_index("c")
        s = lax.axis_index("s")
        pltpu.sync_copy(zrow, acc.at[pl.ds(s * rows_pt, rows_pt)])
        plsc.subcore_barrier()
        sbase = c * e_pad + s * edges_pt
        dbase = s * edges_pt

        def chunk(j, carry):
            pltpu.sync_copy(src2.at[pl.ds(sbase + j * _CH, _CH)], idx_s)
            pltpu.sync_copy(dst_h.at[pl.ds(dbase + j * _CH, _CH)], idx_d)
            pltpu.async_copy(table2.at[idx_s], rows, sem).wait()
            pltpu.sync_copy(rows, acc.at[idx_d], add=True)
            return carry

        lax.fori_loop(0, nchunk, chunk, 0)
        plsc.subcore_barrier()
        sl = pl.ds(s * rows_pt, rows_pt)
        pltpu.sync_copy(acc.at[sl],
                        out2.at[pl.ds(c * n_pad + s * rows_pt, rows_pt)])

    return agg


def _relu(v):
    return jnp.maximum(v, 0.0)


def _rs(deg_col):
    return lax.rsqrt(jnp.maximum(deg_col, 1.0))


def _tc_in(x, hist2, w1, n_pad):
    n, f = x.shape
    h = f // 2
    br = n_pad // _NS

    def body(x_ref, d_ref, w_ref, o_ref):
        so = _rs(d_ref[:, 0:1])
        o_ref[...] = jnp.dot(x_ref[...] * so, w_ref[...],
                             preferred_element_type=F32)

    return pl.pallas_call(
        body,
        grid=(_NC, _NS),
        in_specs=[pl.BlockSpec((br, f), lambda j, i: (i, 0)),
                  pl.BlockSpec((br, h), lambda j, i: (i, 0)),
                  pl.BlockSpec((f, h), lambda j, i: (0, j))],
        out_specs=pl.BlockSpec((br, h), lambda j, i: (j * _NS + i, 0)),
        out_shape=jax.ShapeDtypeStruct((_NC * n_pad, h), F32),
    )(x, hist2, w1)


def _tc_mid(agg2, hist2, b, w2, n_pad):
    h = agg2.shape[1]
    f = 2 * h
    br = n_pad // _NS

    def body(aa_ref, ab_ref, di_ref, do_ref, b_ref, w_ref, o_ref):
        si = _rs(di_ref[:, 0:1])
        so = _rs(do_ref[:, 0:1])
        ha = _relu(aa_ref[...] * si + b_ref[0:1, :h]) * so
        hb = _relu(ab_ref[...] * si + b_ref[0:1, h:]) * so
        o_ref[...] = (jnp.dot(ha, w_ref[:h, :], preferred_element_type=F32)
                      + jnp.dot(hb, w_ref[h:, :], preferred_element_type=F32))

    return pl.pallas_call(
        body,
        grid=(_NC, _NS),
        in_specs=[pl.BlockSpec((br, h), lambda j, i: (i, 0)),
                  pl.BlockSpec((br, h), lambda j, i: (i + _NS, 0)),
                  pl.BlockSpec((br, h), lambda j, i: (i + _NS, 0)),
                  pl.BlockSpec((br, h), lambda j, i: (i, 0)),
                  pl.BlockSpec((1, f), lambda j, i: (0, 0)),
                  pl.BlockSpec((f, h), lambda j, i: (0, j))],
        out_specs=pl.BlockSpec((br, h), lambda j, i: (j * _NS + i, 0)),
        out_shape=jax.ShapeDtypeStruct((_NC * n_pad, h), F32),
    )(agg2, agg2, hist2, hist2, b, w2)


def _tc_out(agg2, hist2, b2, wd1, bd1, wd2, bd2, wfc, bfc, n_pad, n):
    h = agg2.shape[1]
    f = 2 * h
    ncls = wfc.shape[1]
    br = n_pad // _NS

    def body(aa_ref, ab_ref, di_ref, b2_ref, wd1_ref, bd1_ref, wd2_ref,
             bd2_ref, wfc_ref, bfc_ref, lg_ref, xd_ref):
        si = _rs(di_ref[:, 0:1])
        ha = _relu(aa_ref[...] * si + b2_ref[0:1, :h])
        hb = _relu(ab_ref[...] * si + b2_ref[0:1, h:])
        h2 = jnp.concatenate([ha, hb], axis=1)
        t = _relu(jnp.dot(h2, wd1_ref[...], preferred_element_type=F32)
                  + bd1_ref[0:1, :])
        xd_ref[...] = (jnp.dot(t, wd2_ref[...], preferred_element_type=F32)
                       + bd2_ref[0:1, :])
        lg_ref[...] = (jnp.dot(h2, wfc_ref[...], preferred_element_type=F32)
                       + bfc_ref[0:1, :])

    return pl.pallas_call(
        body,
        grid=(_NS,),
        in_specs=[pl.BlockSpec((br, h), lambda i: (i, 0)),
                  pl.BlockSpec((br, h), lambda i: (i + _NS, 0)),
                  pl.BlockSpec((br, h), lambda i: (i + _NS, 0)),
                  pl.BlockSpec((1, f), lambda i: (0, 0)),
                  pl.BlockSpec((f, f), lambda i: (0, 0)),
                  pl.BlockSpec((1, f), lambda i: (0, 0)),
                  pl.BlockSpec((f, f), lambda i: (0, 0)),
                  pl.BlockSpec((1, f), lambda i: (0, 0)),
                  pl.BlockSpec((f, ncls), lambda i: (0, 0)),
                  pl.BlockSpec((1, ncls), lambda i: (0, 0))],
        out_specs=[pl.BlockSpec((br, ncls), lambda i: (i, 0)),
                   pl.BlockSpec((br, f), lambda i: (i, 0))],
        out_shape=[jax.ShapeDtypeStruct((n, ncls), F32),
                   jax.ShapeDtypeStruct((n, f), F32)],
    )(agg2, agg2, hist2, b2, wd1, bd1, wd2, bd2, wfc, bfc)


def kernel(x, edge_index, W1, b1, W2, b2, Wd1, bd1, Wd2, bd2, Wfc, bfc):
    n, f = x.shape
    h = f // 2
    e = edge_index.shape[1]
    gran = _NS * _CH
    e_pad = ((e + gran - 1) // gran) * gran
    n_pad = ((n + 1 + _NS * 8 - 1) // (_NS * 8)) * (_NS * 8)

    src = edge_index[0]
    dst = edge_index[1]
    pad = e_pad - e
    src_h = jnp.concatenate([src, jnp.full((pad,), n, jnp.int32)])
    src_g = jnp.concatenate([src, jnp.zeros((pad,), jnp.int32)])
    dst_h = jnp.concatenate([dst, jnp.full((pad,), n, jnp.int32)])
    idx2 = jnp.concatenate([src_h, dst_h])
    src2 = jnp.concatenate([src_g, src_g + n_pad])
    ones_w = jnp.ones((_CH, h), F32)
    zrow = jnp.zeros((n_pad // _NS, h), F32)

    hist2 = _deg_kernel(n_pad, e_pad, h)(idx2, zrow, ones_w)
    h0 = _tc_in(x, hist2, W1, n_pad)
    a1 = _agg_kernel(n_pad, e_pad, h)(h0, src2, dst_h, zrow)
    h1 = _tc_mid(a1, hist2, b1.reshape(1, f), W2, n_pad)
    a2 = _agg_kernel(n_pad, e_pad, h)(h1, src2, dst_h, zrow)
    logits, x_d = _tc_out(a2, hist2, b2.reshape(1, f),
                          Wd1, bd1.reshape(1, f), Wd2, bd2.reshape(1, f),
                          Wfc, bfc.reshape(1, Wfc.shape[1]), n_pad, n)
    return (logits, x_d)


# speedup vs baseline: 3.8442x; 1.2969x over previous
import functools

import jax
import jax.numpy as jnp
from jax import lax
from jax.experimental import pallas as pl
from jax.experimental.pallas import tpu as pltpu
from jax.experimental.pallas import tpu_sc as plsc

F32 = jnp.float32
_NS = 16
_NC = 2
_CH = 128


@functools.lru_cache(maxsize=None)
def _deg_kernel(n_pad, e_pad, w):
    rows_pt = n_pad // _NS
    edges_pt = e_pad // _NS
    nchunk = edges_pt // _CH
    mesh = plsc.VectorSubcoreMesh(core_axis_name="c", subcore_axis_name="s")

    @functools.partial(
        pl.kernel,
        mesh=mesh,
        out_type=jax.ShapeDtypeStruct((_NC * n_pad, w), F32),
        scratch_types=[pltpu.VMEM((_CH,), jnp.int32),
                       pltpu.VMEM((_CH, w), F32),
                       pltpu.VMEM_SHARED((n_pad, w), F32)],
    )
    def deg(idx2, zeros16, ones16, hist2, idx_v, ones_v, hist):
        c = lax.axis_index("c")
        s = lax.axis_index("s")
        pltpu.sync_copy(zeros16, hist.at[pl.ds(s * rows_pt, rows_pt)])
        pltpu.sync_copy(ones16, ones_v)
        plsc.subcore_barrier()
        base = c * e_pad + s * edges_pt

        def chunk(j, carry):
            pltpu.sync_copy(idx2.at[pl.ds(base + j * _CH, _CH)], idx_v)
            pltpu.sync_copy(ones_v, hist.at[idx_v], add=True)
            return carry

        lax.fori_loop(0, nchunk, chunk, 0)
        plsc.subcore_barrier()
        sl = pl.ds(s * rows_pt, rows_pt)
        pltpu.sync_copy(hist.at[sl],
                        hist2.at[pl.ds(c * n_pad + s * rows_pt, rows_pt)])

    return deg


def _vcopy128(src_ref, soff, dst_ref):
    for k in range(8):
        dst_ref[pl.ds(k * 16, 16)] = src_ref[pl.ds(soff + k * 16, 16)]


@functools.lru_cache(maxsize=None)
def _agg_kernel(n_pad, e_pad, h):
    rows_pt = n_pad // _NS
    edges_pt = e_pad // _NS
    nchunk = edges_pt // _CH
    mesh = plsc.VectorSubcoreMesh(core_axis_name="c", subcore_axis_name="s")

    @functools.partial(
        pl.kernel,
        mesh=mesh,
        out_type=jax.ShapeDtypeStruct((_NC * n_pad, h), F32),
        scratch_types=[pltpu.VMEM(((nchunk + 1) * _CH,), jnp.int32),
                       pltpu.VMEM((_CH,), jnp.int32),
                       pltpu.VMEM((_CH,), jnp.int32),
                       pltpu.VMEM((_CH,), jnp.int32),
                       pltpu.VMEM((_CH,), jnp.int32),
                       pltpu.VMEM((_CH, h), F32),
                       pltpu.VMEM((_CH, h), F32),
                       pltpu.VMEM_SHARED((n_pad, h), F32),
                       pltpu.SemaphoreType.DMA,
                       pltpu.SemaphoreType.DMA,
                       pltpu.SemaphoreType.DMA,
                       pltpu.SemaphoreType.DMA,
                       pltpu.SemaphoreType.DMA,
                       pltpu.SemaphoreType.DMA],
    )
    def agg(table2, src2f, dst_f2, zrow, out2, idx_s, isw_a, isw_b, idw_a,
            idw_b, rows_a, rows_b, acc, sem_a, sem_b, sem_da, sem_db,
            sem_sa, sem_sb):
        c = lax.axis_index("c")
        s = lax.axis_index("s")
        sbase = c * e_pad + s * edges_pt
        dbase = s * edges_pt
        pltpu.sync_copy(src2f.at[pl.ds(sbase, (nchunk + 1) * _CH)], idx_s)
        pltpu.sync_copy(zrow, acc.at[pl.ds(s * rows_pt, rows_pt)])
        plsc.subcore_barrier()

        def gatherw(isw, rows, sem):
            pltpu.async_copy(table2.at[isw], rows, sem)

        def gwaitw(isw, rows, sem):
            pltpu.make_async_copy(table2.at[isw], rows, sem).wait()

        def dload(j, idw, sem):
            pltpu.async_copy(dst_f2.at[pl.ds(dbase + j * _CH, _CH)], idw, sem)

        def dwait(idw, sem):
            pltpu.make_async_copy(dst_f2.at[pl.ds(0, _CH)], idw, sem).wait()

        def scat(rows, idw, sem):
            pltpu.async_copy(rows, acc.at[idw], sem, add=True)

        def swait(rows, idw, sem):
            pltpu.make_async_copy(rows, acc.at[idw], sem).wait()

        _vcopy128(idx_s, 0, isw_a)
        dload(0, idw_a, sem_da)
        gatherw(isw_a, rows_a, sem_a)

        def pair(jj, carry):
            j = jj * 2
            gwaitw(isw_a, rows_a, sem_a)
            dwait(idw_a, sem_da)
            scat(rows_a, idw_a, sem_sa)
            _vcopy128(idx_s, (j + 1) * _CH, isw_b)
            gatherw(isw_b, rows_b, sem_b)
            dload(j + 1, idw_b, sem_db)
            swait(rows_a, idw_a, sem_sa)
            gwaitw(isw_b, rows_b, sem_b)
            dwait(idw_b, sem_db)
            scat(rows_b, idw_b, sem_sb)
            _vcopy128(idx_s, (j + 2) * _CH, isw_a)
            gatherw(isw_a, rows_a, sem_a)
            dload(j + 2, idw_a, sem_da)
            swait(rows_b, idw_b, sem_sb)
            return carry

        lax.fori_loop(0, nchunk // 2, pair, 0)
        if nchunk % 2 == 1:
            gwaitw(isw_a, rows_a, sem_a)
            dwait(idw_a, sem_da)
            scat(rows_a, idw_a, sem_sa)
            swait(rows_a, idw_a, sem_sa)
        else:
            gwaitw(isw_a, rows_a, sem_a)
            dwait(idw_a, sem_da)
        plsc.subcore_barrier()
        sl = pl.ds(s * rows_pt, rows_pt)
        pltpu.sync_copy(acc.at[sl],
                        out2.at[pl.ds(c * n_pad + s * rows_pt, rows_pt)])

    return agg


def _relu(v):
    return jnp.maximum(v, 0.0)


def _rs(deg_col):
    return lax.rsqrt(jnp.maximum(deg_col, 1.0))


def _tc_in(x, hist2, w1, n_pad):
    n, f = x.shape
    h = f // 2
    br = n_pad // _NS

    def body(x_ref, d_ref, w_ref, o_ref):
        so = _rs(d_ref[:, 0:1])
        o_ref[...] = jnp.dot(x_ref[...] * so, w_ref[...],
                             preferred_element_type=F32)

    return pl.pallas_call(
        body,
        grid=(_NC, _NS),
        in_specs=[pl.BlockSpec((br, f), lambda j, i: (i, 0)),
                  pl.BlockSpec((br, h), lambda j, i: (i, 0)),
                  pl.BlockSpec((f, h), lambda j, i: (0, j))],
        out_specs=pl.BlockSpec((br, h), lambda j, i: (j * _NS + i, 0)),
        out_shape=jax.ShapeDtypeStruct((_NC * n_pad, h), F32),
    )(x, hist2, w1)


def _tc_mid(agg2, hist2, b, w2, n_pad):
    h = agg2.shape[1]
    f = 2 * h
    br = n_pad // _NS

    def body(aa_ref, ab_ref, di_ref, do_ref, b_ref, w_ref, o_ref):
        si = _rs(di_ref[:, 0:1])
        so = _rs(do_ref[:, 0:1])
        ha = _relu(aa_ref[...] * si + b_ref[0:1, :h]) * so
        hb = _relu(ab_ref[...] * si + b_ref[0:1, h:]) * so
        o_ref[...] = (jnp.dot(ha, w_ref[:h, :], preferred_element_type=F32)
                      + jnp.dot(hb, w_ref[h:, :], preferred_element_type=F32))

    return pl.pallas_call(
        body,
        grid=(_NC, _NS),
        in_specs=[pl.BlockSpec((br, h), lambda j, i: (i, 0)),
                  pl.BlockSpec((br, h), lambda j, i: (i + _NS, 0)),
                  pl.BlockSpec((br, h), lambda j, i: (i + _NS, 0)),
                  pl.BlockSpec((br, h), lambda j, i: (i, 0)),
                  pl.BlockSpec((1, f), lambda j, i: (0, 0)),
                  pl.BlockSpec((f, h), lambda j, i: (0, j))],
        out_specs=pl.BlockSpec((br, h), lambda j, i: (j * _NS + i, 0)),
        out_shape=jax.ShapeDtypeStruct((_NC * n_pad, h), F32),
    )(agg2, agg2, hist2, hist2, b, w2)


def _tc_out(agg2, hist2, b2, wd1, bd1, wd2, bd2, wfc, bfc, n_pad, n):
    h = agg2.shape[1]
    f = 2 * h
    ncls = wfc.shape[1]
    br = n_pad // _NS

    def body(aa_ref, ab_ref, di_ref, b2_ref, wd1_ref, bd1_ref, wd2_ref,
             bd2_ref, wfc_ref, bfc_ref, lg_ref, xd_ref):
        si = _rs(di_ref[:, 0:1])
        ha = _relu(aa_ref[...] * si + b2_ref[0:1, :h])
        hb = _relu(ab_ref[...] * si + b2_ref[0:1, h:])
        h2 = jnp.concatenate([ha, hb], axis=1)
        t = _relu(jnp.dot(h2, wd1_ref[...], preferred_element_type=F32)
                  + bd1_ref[0:1, :])
        xd_ref[...] = (jnp.dot(t, wd2_ref[...], preferred_element_type=F32)
                       + bd2_ref[0:1, :])
        lg_ref[...] = (jnp.dot(h2, wfc_ref[...], preferred_element_type=F32)
                       + bfc_ref[0:1, :])

    return pl.pallas_call(
        body,
        grid=(_NS,),
        in_specs=[pl.BlockSpec((br, h), lambda i: (i, 0)),
                  pl.BlockSpec((br, h), lambda i: (i + _NS, 0)),
                  pl.BlockSpec((br, h), lambda i: (i + _NS, 0)),
                  pl.BlockSpec((1, f), lambda i: (0, 0)),
                  pl.BlockSpec((f, f), lambda i: (0, 0)),
                  pl.BlockSpec((1, f), lambda i: (0, 0)),
                  pl.BlockSpec((f, f), lambda i: (0, 0)),
                  pl.BlockSpec((1, f), lambda i: (0, 0)),
                  pl.BlockSpec((f, ncls), lambda i: (0, 0)),
                  pl.BlockSpec((1, ncls), lambda i: (0, 0))],
        out_specs=[pl.BlockSpec((br, ncls), lambda i: (i, 0)),
                   pl.BlockSpec((br, f), lambda i: (i, 0))],
        out_shape=[jax.ShapeDtypeStruct((n, ncls), F32),
                   jax.ShapeDtypeStruct((n, f), F32)],
    )(agg2, agg2, hist2, b2, wd1, bd1, wd2, bd2, wfc, bfc)


def kernel(x, edge_index, W1, b1, W2, b2, Wd1, bd1, Wd2, bd2, Wfc, bfc):
    n, f = x.shape
    h = f // 2
    e = edge_index.shape[1]
    gran = _NS * _CH
    e_pad = ((e + gran - 1) // gran) * gran
    n_pad = ((n + 1 + _NS * 8 - 1) // (_NS * 8)) * (_NS * 8)

    src = edge_index[0]
    dst = edge_index[1]
    pad = e_pad - e
    src_h = jnp.concatenate([src, jnp.full((pad,), n, jnp.int32)])
    src_g = jnp.concatenate([src, jnp.zeros((pad,), jnp.int32)])
    dst_h = jnp.concatenate([dst, jnp.full((pad,), n, jnp.int32)])
    idx2 = jnp.concatenate([src_h, dst_h])
    src2f = jnp.concatenate([src_g, src_g + n_pad,
                             jnp.zeros((_CH,), jnp.int32)])
    dst_f2 = jnp.concatenate([dst_h, jnp.zeros((_CH,), jnp.int32)])
    ones_w = jnp.ones((_CH, h), F32)
    zrow = jnp.zeros((n_pad // _NS, h), F32)

    hist2 = _deg_kernel(n_pad, e_pad, h)(idx2, zrow, ones_w)
    h0 = _tc_in(x, hist2, W1, n_pad)
    a1 = _agg_kernel(n_pad, e_pad, h)(h0, src2f, dst_f2, zrow)
    h1 = _tc_mid(a1, hist2, b1.reshape(1, f), W2, n_pad)
    a2 = _agg_kernel(n_pad, e_pad, h)(h1, src2f, dst_f2, zrow)
    logits, x_d = _tc_out(a2, hist2, b2.reshape(1, f),
                          Wd1, bd1.reshape(1, f), Wd2, bd2.reshape(1, f),
                          Wfc, bfc.reshape(1, Wfc.shape[1]), n_pad, n)
    return (logits, x_d)


# speedup vs baseline: 3.9322x; 1.0229x over previous
import functools

import jax
import jax.numpy as jnp
from jax import lax
from jax.experimental import pallas as pl
from jax.experimental.pallas import tpu as pltpu
from jax.experimental.pallas import tpu_sc as plsc

F32 = jnp.float32
_NS = 16
_NC = 2
_CH = 128


@functools.lru_cache(maxsize=None)
def _deg_kernel(n_pad, e_pad, w):
    rows_pt = n_pad // _NS
    edges_pt = e_pad // _NS
    nchunk = edges_pt // _CH
    mesh = plsc.VectorSubcoreMesh(core_axis_name="c", subcore_axis_name="s")

    @functools.partial(
        pl.kernel,
        mesh=mesh,
        out_type=jax.ShapeDtypeStruct((_NC * n_pad, w), F32),
        scratch_types=[pltpu.VMEM((_CH,), jnp.int32),
                       pltpu.VMEM((_CH,), jnp.int32),
                       pltpu.VMEM((_CH, w), F32),
                       pltpu.VMEM_SHARED((n_pad, w), F32),
                       pltpu.SemaphoreType.DMA,
                       pltpu.SemaphoreType.DMA],
    )
    def deg(idx2p, zeros16, ones16, hist2, idxv_a, idxv_b, ones_v, hist,
            sem_ia, sem_ib):
        c = lax.axis_index("c")
        s = lax.axis_index("s")
        pltpu.sync_copy(zeros16, hist.at[pl.ds(s * rows_pt, rows_pt)])
        pltpu.sync_copy(ones16, ones_v)
        plsc.subcore_barrier()
        base = c * e_pad + s * edges_pt

        def iload(j, idxv, sem):
            pltpu.async_copy(idx2p.at[pl.ds(base + j * _CH, _CH)], idxv, sem)

        def iwait(idxv, sem):
            pltpu.make_async_copy(idx2p.at[pl.ds(0, _CH)], idxv, sem).wait()

        iload(0, idxv_a, sem_ia)
        iload(1, idxv_b, sem_ib)

        def pair(jj, carry):
            j = jj * 2
            iwait(idxv_a, sem_ia)
            pltpu.sync_copy(ones_v, hist.at[idxv_a], add=True)
            iload(j + 2, idxv_a, sem_ia)
            iwait(idxv_b, sem_ib)
            pltpu.sync_copy(ones_v, hist.at[idxv_b], add=True)
            iload(j + 3, idxv_b, sem_ib)
            return carry

        lax.fori_loop(0, nchunk // 2, pair, 0)
        if nchunk % 2 == 1:
            iwait(idxv_a, sem_ia)
            pltpu.sync_copy(ones_v, hist.at[idxv_a], add=True)
            iwait(idxv_b, sem_ib)
        else:
            iwait(idxv_a, sem_ia)
            iwait(idxv_b, sem_ib)
        plsc.subcore_barrier()
        sl = pl.ds(s * rows_pt, rows_pt)
        pltpu.sync_copy(hist.at[sl],
                        hist2.at[pl.ds(c * n_pad + s * rows_pt, rows_pt)])

    return deg


def _vcopy128(src_ref, soff, dst_ref):
    for k in range(8):
        dst_ref[pl.ds(k * 16, 16)] = src_ref[pl.ds(soff + k * 16, 16)]


@functools.lru_cache(maxsize=None)
def _agg_kernel(n_pad, e_pad, h):
    rows_pt = n_pad // _NS
    edges_pt = e_pad // _NS
    nchunk = edges_pt // _CH
    mesh = plsc.VectorSubcoreMesh(core_axis_name="c", subcore_axis_name="s")

    @functools.partial(
        pl.kernel,
        mesh=mesh,
        out_type=jax.ShapeDtypeStruct((_NC * n_pad, h), F32),
        scratch_types=[pltpu.VMEM(((nchunk + 1) * _CH,), jnp.int32),
                       pltpu.VMEM((_CH,), jnp.int32),
                       pltpu.VMEM((_CH,), jnp.int32),
                       pltpu.VMEM((_CH,), jnp.int32),
                       pltpu.VMEM((_CH,), jnp.int32),
                       pltpu.VMEM((_CH, h), F32),
                       pltpu.VMEM((_CH, h), F32),
                       pltpu.VMEM_SHARED((n_pad, h), F32),
                       pltpu.SemaphoreType.DMA,
                       pltpu.SemaphoreType.DMA,
                       pltpu.SemaphoreType.DMA,
                       pltpu.SemaphoreType.DMA,
                       pltpu.SemaphoreType.DMA,
                       pltpu.SemaphoreType.DMA],
    )
    def agg(table2, src2f, dst_f2, zrow, out2, idx_s, isw_a, isw_b, idw_a,
            idw_b, rows_a, rows_b, acc, sem_a, sem_b, sem_da, sem_db,
            sem_sa, sem_sb):
        c = lax.axis_index("c")
        s = lax.axis_index("s")
        sbase = c * e_pad + s * edges_pt
        dbase = s * edges_pt
        pltpu.sync_copy(src2f.at[pl.ds(sbase, (nchunk + 1) * _CH)], idx_s)
        pltpu.sync_copy(zrow, acc.at[pl.ds(s * rows_pt, rows_pt)])
        plsc.subcore_barrier()

        def gatherw(isw, rows, sem):
            pltpu.async_copy(table2.at[isw], rows, sem)

        def gwaitw(isw, rows, sem):
            pltpu.make_async_copy(table2.at[isw], rows, sem).wait()

        def dload(j, idw, sem):
            pltpu.async_copy(dst_f2.at[pl.ds(dbase + j * _CH, _CH)], idw, sem)

        def dwait(idw, sem):
            pltpu.make_async_copy(dst_f2.at[pl.ds(0, _CH)], idw, sem).wait()

        def scat(rows, idw, sem):
            pltpu.async_copy(rows, acc.at[idw], sem, add=True)

        def swait(rows, idw, sem):
            pltpu.make_async_copy(rows, acc.at[idw], sem).wait()

        _vcopy128(idx_s, 0, isw_a)
        dload(0, idw_a, sem_da)
        gatherw(isw_a, rows_a, sem_a)

        def pair(jj, carry):
            j = jj * 2
            gwaitw(isw_a, rows_a, sem_a)
            dwait(idw_a, sem_da)
            scat(rows_a, idw_a, sem_sa)
            _vcopy128(idx_s, (j + 1) * _CH, isw_b)
            gatherw(isw_b, rows_b, sem_b)
            dload(j + 1, idw_b, sem_db)
            swait(rows_a, idw_a, sem_sa)
            gwaitw(isw_b, rows_b, sem_b)
            dwait(idw_b, sem_db)
            scat(rows_b, idw_b, sem_sb)
            _vcopy128(idx_s, (j + 2) * _CH, isw_a)
            gatherw(isw_a, rows_a, sem_a)
            dload(j + 2, idw_a, sem_da)
            swait(rows_b, idw_b, sem_sb)
            return carry

        lax.fori_loop(0, nchunk // 2, pair, 0)
        if nchunk % 2 == 1:
            gwaitw(isw_a, rows_a, sem_a)
            dwait(idw_a, sem_da)
            scat(rows_a, idw_a, sem_sa)
            swait(rows_a, idw_a, sem_sa)
        else:
            gwaitw(isw_a, rows_a, sem_a)
            dwait(idw_a, sem_da)
        plsc.subcore_barrier()
        sl = pl.ds(s * rows_pt, rows_pt)
        pltpu.sync_copy(acc.at[sl],
                        out2.at[pl.ds(c * n_pad + s * rows_pt, rows_pt)])

    return agg


def _relu(v):
    return jnp.maximum(v, 0.0)


def _rs(deg_col):
    return lax.rsqrt(jnp.maximum(deg_col, 1.0))


def _tc_in(x, hist2, w1, n_pad):
    n, f = x.shape
    h = f // 2
    br = n_pad // _NS

    def body(x_ref, d_ref, w_ref, o_ref):
        so = _rs(d_ref[:, 0:1])
        o_ref[...] = jnp.dot(x_ref[...] * so, w_ref[...],
                             preferred_element_type=F32)

    return pl.pallas_call(
        body,
        grid=(_NC, _NS),
        in_specs=[pl.BlockSpec((br, f), lambda j, i: (i, 0)),
                  pl.BlockSpec((br, h), lambda j, i: (i, 0)),
                  pl.BlockSpec((f, h), lambda j, i: (0, j))],
        out_specs=pl.BlockSpec((br, h), lambda j, i: (j * _NS + i, 0)),
        out_shape=jax.ShapeDtypeStruct((_NC * n_pad, h), F32),
    )(x, hist2, w1)


def _tc_mid(agg2, hist2, b, w2, n_pad):
    h = agg2.shape[1]
    f = 2 * h
    br = n_pad // _NS

    def body(aa_ref, ab_ref, di_ref, do_ref, b_ref, w_ref, o_ref):
        si = _rs(di_ref[:, 0:1])
        so = _rs(do_ref[:, 0:1])
        ha = _relu(aa_ref[...] * si + b_ref[0:1, :h]) * so
        hb = _relu(ab_ref[...] * si + b_ref[0:1, h:]) * so
        o_ref[...] = (jnp.dot(ha, w_ref[:h, :], preferred_element_type=F32)
                      + jnp.dot(hb, w_ref[h:, :], preferred_element_type=F32))

    return pl.pallas_call(
        body,
        grid=(_NC, _NS),
        in_specs=[pl.BlockSpec((br, h), lambda j, i: (i, 0)),
                  pl.BlockSpec((br, h), lambda j, i: (i + _NS, 0)),
                  pl.BlockSpec((br, h), lambda j, i: (i + _NS, 0)),
                  pl.BlockSpec((br, h), lambda j, i: (i, 0)),
                  pl.BlockSpec((1, f), lambda j, i: (0, 0)),
                  pl.BlockSpec((f, h), lambda j, i: (0, j))],
        out_specs=pl.BlockSpec((br, h), lambda j, i: (j * _NS + i, 0)),
        out_shape=jax.ShapeDtypeStruct((_NC * n_pad, h), F32),
    )(agg2, agg2, hist2, hist2, b, w2)


def _tc_out(agg2, hist2, b2, wd1, bd1, wd2, bd2, wfc, bfc, n_pad, n):
    h = agg2.shape[1]
    f = 2 * h
    ncls = wfc.shape[1]
    br = n_pad // _NS

    def body(aa_ref, ab_ref, di_ref, b2_ref, wd1_ref, bd1_ref, wd2_ref,
             bd2_ref, wfc_ref, bfc_ref, lg_ref, xd_ref):
        si = _rs(di_ref[:, 0:1])
        ha = _relu(aa_ref[...] * si + b2_ref[0:1, :h])
        hb = _relu(ab_ref[...] * si + b2_ref[0:1, h:])
        h2 = jnp.concatenate([ha, hb], axis=1)
        t = _relu(jnp.dot(h2, wd1_ref[...], preferred_element_type=F32)
                  + bd1_ref[0:1, :])
        xd_ref[...] = (jnp.dot(t, wd2_ref[...], preferred_element_type=F32)
                       + bd2_ref[0:1, :])
        lg_ref[...] = (jnp.dot(h2, wfc_ref[...], preferred_element_type=F32)
                       + bfc_ref[0:1, :])

    return pl.pallas_call(
        body,
        grid=(_NS,),
        in_specs=[pl.BlockSpec((br, h), lambda i: (i, 0)),
                  pl.BlockSpec((br, h), lambda i: (i + _NS, 0)),
                  pl.BlockSpec((br, h), lambda i: (i + _NS, 0)),
                  pl.BlockSpec((1, f), lambda i: (0, 0)),
                  pl.BlockSpec((f, f), lambda i: (0, 0)),
                  pl.BlockSpec((1, f), lambda i: (0, 0)),
                  pl.BlockSpec((f, f), lambda i: (0, 0)),
                  pl.BlockSpec((1, f), lambda i: (0, 0)),
                  pl.BlockSpec((f, ncls), lambda i: (0, 0)),
                  pl.BlockSpec((1, ncls), lambda i: (0, 0))],
        out_specs=[pl.BlockSpec((br, ncls), lambda i: (i, 0)),
                   pl.BlockSpec((br, f), lambda i: (i, 0))],
        out_shape=[jax.ShapeDtypeStruct((n, ncls), F32),
                   jax.ShapeDtypeStruct((n, f), F32)],
    )(agg2, agg2, hist2, b2, wd1, bd1, wd2, bd2, wfc, bfc)


def kernel(x, edge_index, W1, b1, W2, b2, Wd1, bd1, Wd2, bd2, Wfc, bfc):
    n, f = x.shape
    h = f // 2
    e = edge_index.shape[1]
    gran = _NS * _CH
    e_pad = ((e + gran - 1) // gran) * gran
    n_pad = ((n + 1 + _NS * 8 - 1) // (_NS * 8)) * (_NS * 8)

    src = edge_index[0]
    dst = edge_index[1]
    pad = e_pad - e
    src_h = jnp.concatenate([src, jnp.full((pad,), n, jnp.int32)])
    src_g = jnp.concatenate([src, jnp.zeros((pad,), jnp.int32)])
    dst_h = jnp.concatenate([dst, jnp.full((pad,), n, jnp.int32)])
    idx2 = jnp.concatenate([src_h, dst_h,
                            jnp.zeros((2 * _CH,), jnp.int32)])
    src2f = jnp.concatenate([src_g, src_g + n_pad,
                             jnp.zeros((_CH,), jnp.int32)])
    dst_f2 = jnp.concatenate([dst_h, jnp.zeros((_CH,), jnp.int32)])
    ones_w = jnp.ones((_CH, h), F32)
    zrow = jnp.zeros((n_pad // _NS, h), F32)

    hist2 = _deg_kernel(n_pad, e_pad, h)(idx2, zrow, ones_w)
    h0 = _tc_in(x, hist2, W1, n_pad)
    a1 = _agg_kernel(n_pad, e_pad, h)(h0, src2f, dst_f2, zrow)
    h1 = _tc_mid(a1, hist2, b1.reshape(1, f), W2, n_pad)
    a2 = _agg_kernel(n_pad, e_pad, h)(h1, src2f, dst_f2, zrow)
    logits, x_d = _tc_out(a2, hist2, b2.reshape(1, f),
                          Wd1, bd1.reshape(1, f), Wd2, bd2.reshape(1, f),
                          Wfc, bfc.reshape(1, Wfc.shape[1]), n_pad, n)
    return (logits, x_d)
